# Initial kernel scaffold; baseline (speedup 1.0000x reference)
import jax
import jax.numpy as jnp
from jax.experimental import pallas as pl


def kernel(item_id, sims):
    raise NotImplementedError("write your pallas kernel here")


# speedup vs baseline: 15.9228x; 15.9228x over previous
import functools

import jax
import jax.numpy as jnp
from jax import lax
from jax.experimental import pallas as pl
from jax.experimental.pallas import tpu as pltpu
from jax.experimental.pallas import tpu_sc as plsc

_N_ITEMS = 16384
_KNN = 100
_BATCH = 4096
_L = 16
_NC = 2
_NS = 16
_NW = _NC * _NS
_RPW = _BATCH // _NW
_NV = _N_ITEMS // _L
_TOPB = 112
_NSLOT = _TOPB // _L
_NEG = -3.4028234663852886e38


def _topk_body(iid_hbm, sims_hbm, ov_hbm, oi_hbm,
               ids_v, row_v, tv_v, ti_v, sv_v, si_v, sem):
    wid = lax.axis_index("s") * _NC + lax.axis_index("c")
    base = wid * _RPW
    pltpu.sync_copy(iid_hbm.at[pl.ds(base, _RPW)], ids_v)
    iota = lax.iota(jnp.int32, _L)

    def row_body(r, carry):
        pltpu.async_copy(sims_hbm.at[ids_v.at[r]], row_v, sem).wait()
        for j in range(_NSLOT):
            tv_v[pl.ds(_L * j, _L)] = jnp.full((_L,), _NEG, jnp.float32)
            ti_v[pl.ds(_L * j, _L)] = jnp.zeros((_L,), jnp.int32)

        def vec_body(i, thrv):
            v = row_v[0, pl.ds(i * _L, _L)]
            m = v > thrv

            def do_insert():
                ck, cv = plsc.sort_key_val(v, iota + i * _L,
                                           descending=True)
                last_hk = None
                for j in range(_NSLOT):
                    ak = tv_v[pl.ds(_L * j, _L)]
                    ai = ti_v[pl.ds(_L * j, _L)]
                    rk = jnp.flip(ck, 0)
                    rv = jnp.flip(cv, 0)
                    c = ak >= rk
                    hk = jnp.where(c, ak, rk)
                    hv = jnp.where(c, ai, rv)
                    lk = jnp.where(c, rk, ak)
                    lv = jnp.where(c, rv, ai)
                    hk, hv = plsc.sort_key_val(hk, hv, descending=True)
                    tv_v[pl.ds(_L * j, _L)] = hk
                    ti_v[pl.ds(_L * j, _L)] = hv
                    if j < _NSLOT - 1:
                        ck, cv = plsc.sort_key_val(lk, lv, descending=True)
                    last_hk = hk
                last_lane = jnp.full((_L,), _L - 1, jnp.int32)
                return last_hk.at[last_lane].get(mode="promise_in_bounds")

            return lax.cond(jnp.any(m), do_insert, lambda: thrv)

        thr0 = jnp.full((_L,), _NEG, jnp.float32)
        lax.fori_loop(0, _NV, vec_body, thr0)

        def _bcast(x, lane):
            return x.at[jnp.full((_L,), lane, jnp.int32)].get(
                mode="promise_in_bounds")

        vs = [tv_v[pl.ds(_L * j, _L)] for j in range(_NSLOT)]
        ix = [ti_v[pl.ds(_L * j, _L)] for j in range(_NSLOT)]
        even = (iota & 1) == 0
        perm_a = iota ^ 1
        perm_b = jnp.clip(jnp.where(even, iota - 1, iota + 1), 0, _L - 1)
        for phase in range(4):
            if phase % 2 == 0:
                for j in range(_NSLOT):
                    pv = vs[j].at[perm_a].get(mode="promise_in_bounds")
                    pi = ix[j].at[perm_a].get(mode="promise_in_bounds")
                    veq = vs[j] == pv
                    upd = jnp.where(even, jnp.minimum(ix[j], pi),
                                    jnp.maximum(ix[j], pi))
                    ix[j] = jnp.where(veq, upd, ix[j])
            else:
                ix0 = list(ix)
                for j in range(_NSLOT):
                    pv = vs[j].at[perm_b].get(mode="promise_in_bounds")
                    pi = ix0[j].at[perm_b].get(mode="promise_in_bounds")
                    elig = jnp.ones((_L,), jnp.bool_)
                    if j > 0:
                        at0 = iota == 0
                        pv = jnp.where(at0, _bcast(vs[j - 1], _L - 1), pv)
                        pi = jnp.where(at0, _bcast(ix0[j - 1], _L - 1), pi)
                    else:
                        elig = elig & (iota != 0)
                    if j < _NSLOT - 1:
                        at15 = iota == _L - 1
                        pv = jnp.where(at15, _bcast(vs[j + 1], 0), pv)
                        pi = jnp.where(at15, _bcast(ix0[j + 1], 0), pi)
                    else:
                        elig = elig & (iota != _L - 1)
                    veq = (vs[j] == pv) & elig
                    upd = jnp.where(~even, jnp.minimum(ix0[j], pi),
                                    jnp.maximum(ix0[j], pi))
                    ix[j] = jnp.where(veq, upd, ix0[j])
        for j in range(_NSLOT):
            ti_v[pl.ds(_L * j, _L)] = ix[j]

        for j in range(_KNN // _L):
            sv_v[r, pl.ds(_L * j, _L)] = tv_v[pl.ds(_L * j, _L)]
            si_v[r, pl.ds(_L * j, _L)] = ti_v[pl.ds(_L * j, _L)]
        sv_v[r, pl.ds(_KNN - _L, _L)] = tv_v[pl.ds(_KNN - _L, _L)]
        si_v[r, pl.ds(_KNN - _L, _L)] = ti_v[pl.ds(_KNN - _L, _L)]
        return carry

    lax.fori_loop(0, _RPW, row_body, 0)
    pltpu.sync_copy(sv_v, ov_hbm.at[pl.ds(base, _RPW)])
    pltpu.sync_copy(si_v, oi_hbm.at[pl.ds(base, _RPW)])


_sc_topk = functools.partial(
    pl.kernel,
    out_type=[
        jax.ShapeDtypeStruct((_BATCH, _KNN), jnp.float32),
        jax.ShapeDtypeStruct((_BATCH, _KNN), jnp.int32),
    ],
    mesh=plsc.VectorSubcoreMesh(core_axis_name="c", subcore_axis_name="s"),
    compiler_params=pltpu.CompilerParams(needs_layout_passes=False),
    scratch_types=[
        pltpu.VMEM((_RPW, 1), jnp.int32),
        pltpu.VMEM((1, _N_ITEMS), jnp.float32),
        pltpu.VMEM((_TOPB,), jnp.float32),
        pltpu.VMEM((_TOPB,), jnp.int32),
        pltpu.VMEM((_RPW, _KNN), jnp.float32),
        pltpu.VMEM((_RPW, _KNN), jnp.int32),
        pltpu.SemaphoreType.DMA,
    ],
)(_topk_body)


def kernel(item_id, sims):
    iid = item_id.astype(jnp.int32).reshape(_BATCH, 1)
    vals, idxs = _sc_topk(iid, sims)
    return vals, idxs


# speedup vs baseline: 121.8166x; 7.6505x over previous
import functools

import jax
import jax.numpy as jnp
from jax import lax
from jax.experimental import pallas as pl
from jax.experimental.pallas import tpu as pltpu
from jax.experimental.pallas import tpu_sc as plsc

_N_ITEMS = 16384
_KNN = 100
_BATCH = 4096
_L = 16
_NC = 2
_NS = 16
_NW = _NC * _NS
_RPW = _BATCH // _NW
_NV = _N_ITEMS // _L
_GRP = 4
_NG = _NV // _GRP
_TOPB = 112
_NSLOT = _TOPB // _L
_CAP = _N_ITEMS + _L
_NEG = -3.4028234663852886e38
_THRESH = 2.21


def _topk_body(iid_hbm, sims_hbm, ov_hbm, oi_hbm,
               ids_v, row_v, cv_v, ci_v, tv_v, ti_v, sv_v, si_v,
               sem0, sem1):
    wid = lax.axis_index("s") * _NC + lax.axis_index("c")
    base = wid * _RPW
    pltpu.sync_copy(iid_hbm.at[pl.ds(base, _RPW)], ids_v)
    iota = lax.iota(jnp.int32, _L)
    even = (iota & 1) == 0
    perm_a = iota ^ 1
    perm_b = jnp.clip(jnp.where(even, iota - 1, iota + 1), 0, _L - 1)

    def _bcast(x, lane):
        return x.at[jnp.full((_L,), lane, jnp.int32)].get(
            mode="promise_in_bounds")

    def process_row(buf, rr):
        for j in range(_NSLOT):
            tv_v[pl.ds(_L * j, _L)] = jnp.full((_L,), _NEG, jnp.float32)
            ti_v[pl.ds(_L * j, _L)] = jnp.zeros((_L,), jnp.int32)

        def pa_body(i, pos):
            b0 = i * (_GRP * _L)
            vs = [row_v[buf, 0, pl.ds(b0 + _L * q, _L)] for q in range(_GRP)]
            ms = [v >= _THRESH for v in vs]
            cs = [plsc.all_reduce_population_count(m)[0] for m in ms]
            p = pos
            for q in range(_GRP):
                plsc.store_compressed(cv_v.at[pl.ds(p, _L)], vs[q],
                                      mask=ms[q])
                plsc.store_compressed(ci_v.at[pl.ds(p, _L)],
                                      iota + (b0 + _L * q), mask=ms[q])
                p = p + cs[q]
            return p

        n = lax.fori_loop(0, _NG, pa_body, jnp.int32(0))
        cv_v[pl.ds(n, _L)] = jnp.full((_L,), _NEG, jnp.float32)
        ci_v[pl.ds(n, _L)] = jnp.zeros((_L,), jnp.int32)

        def fallback():
            def cp(i, _):
                cv_v[pl.ds(i * _L, _L)] = row_v[buf, 0, pl.ds(i * _L, _L)]
                ci_v[pl.ds(i * _L, _L)] = iota + i * _L
                return 0

            lax.fori_loop(0, _NV, cp, 0)
            return jnp.int32(_N_ITEMS)

        n2 = lax.cond(n < _TOPB, fallback, lambda: n)
        nvec = (n2 + _L - 1) // _L

        def pb_body(i, thrv):
            v = cv_v[pl.ds(i * _L, _L)]
            vi = ci_v[pl.ds(i * _L, _L)]
            m = v > thrv

            def do_insert():
                ck, cvp = plsc.sort_key_val(v, vi, descending=True)
                cur_k, cur_v = ck, cvp
                last_hk = None
                for j in range(_NSLOT):
                    ak = tv_v[pl.ds(_L * j, _L)]
                    ai = ti_v[pl.ds(_L * j, _L)]
                    rk = jnp.flip(cur_k, 0)
                    rv = jnp.flip(cur_v, 0)
                    c = ak >= rk
                    hk = jnp.where(c, ak, rk)
                    hv = jnp.where(c, ai, rv)
                    lk = jnp.where(c, rk, ak)
                    lv = jnp.where(c, rv, ai)
                    hk, hv = plsc.sort_key_val(hk, hv, descending=True)
                    tv_v[pl.ds(_L * j, _L)] = hk
                    ti_v[pl.ds(_L * j, _L)] = hv
                    if j < _NSLOT - 1:
                        cur_k, cur_v = plsc.sort_key_val(lk, lv,
                                                         descending=True)
                    last_hk = hk
                return _bcast(last_hk, _L - 1)

            return lax.cond(jnp.any(m), do_insert, lambda: thrv)

        thr0 = jnp.full((_L,), _NEG, jnp.float32)
        lax.fori_loop(0, nvec, pb_body, thr0)

        vs = [tv_v[pl.ds(_L * j, _L)] for j in range(_NSLOT)]
        ix = [ti_v[pl.ds(_L * j, _L)] for j in range(_NSLOT)]
        for phase in range(4):
            if phase % 2 == 0:
                for j in range(_NSLOT):
                    pv = vs[j].at[perm_a].get(mode="promise_in_bounds")
                    pi = ix[j].at[perm_a].get(mode="promise_in_bounds")
                    veq = vs[j] == pv
                    upd = jnp.where(even, jnp.minimum(ix[j], pi),
                                    jnp.maximum(ix[j], pi))
                    ix[j] = jnp.where(veq, upd, ix[j])
            else:
                ix0 = list(ix)
                for j in range(_NSLOT):
                    pv = vs[j].at[perm_b].get(mode="promise_in_bounds")
                    pi = ix0[j].at[perm_b].get(mode="promise_in_bounds")
                    elig = jnp.ones((_L,), jnp.bool_)
                    if j > 0:
                        at0 = iota == 0
                        pv = jnp.where(at0, _bcast(vs[j - 1], _L - 1), pv)
                        pi = jnp.where(at0, _bcast(ix0[j - 1], _L - 1), pi)
                    else:
                        elig = elig & (iota != 0)
                    if j < _NSLOT - 1:
                        at15 = iota == _L - 1
                        pv = jnp.where(at15, _bcast(vs[j + 1], 0), pv)
                        pi = jnp.where(at15, _bcast(ix0[j + 1], 0), pi)
                    else:
                        elig = elig & (iota != _L - 1)
                    veq = (vs[j] == pv) & elig
                    upd = jnp.where(~even, jnp.minimum(ix0[j], pi),
                                    jnp.maximum(ix0[j], pi))
                    ix[j] = jnp.where(veq, upd, ix0[j])

        ti_v[pl.ds(_L * (_NSLOT - 2), _L)] = ix[_NSLOT - 2]
        ti_v[pl.ds(_L * (_NSLOT - 1), _L)] = ix[_NSLOT - 1]
        for j in range(_KNN // _L):
            sv_v[rr, pl.ds(_L * j, _L)] = vs[j]
            si_v[rr, pl.ds(_L * j, _L)] = ix[j]
        sv_v[rr, pl.ds(_KNN - _L, _L)] = tv_v[pl.ds(_KNN - _L, _L)]
        si_v[rr, pl.ds(_KNN - _L, _L)] = ti_v[pl.ds(_KNN - _L, _L)]

    def issue(rr, buf, sem):
        pltpu.async_copy(sims_hbm.at[ids_v.at[rr]], row_v.at[buf], sem)

    def wait(rr, buf, sem):
        pltpu.make_async_copy(sims_hbm.at[ids_v.at[rr]],
                              row_v.at[buf], sem).wait()

    issue(0, 0, sem0)

    def pair_body(k, carry):
        r0 = 2 * k
        issue(r0 + 1, 1, sem1)
        wait(r0, 0, sem0)
        process_row(0, r0)

        @pl.when(k < _RPW // 2 - 1)
        def _():
            issue(r0 + 2, 0, sem0)

        wait(r0 + 1, 1, sem1)
        process_row(1, r0 + 1)
        return carry

    lax.fori_loop(0, _RPW // 2, pair_body, 0)

    pltpu.sync_copy(sv_v, ov_hbm.at[pl.ds(base, _RPW)])
    pltpu.sync_copy(si_v, oi_hbm.at[pl.ds(base, _RPW)])


_sc_topk = functools.partial(
    pl.kernel,
    out_type=[
        jax.ShapeDtypeStruct((_BATCH, _KNN), jnp.float32),
        jax.ShapeDtypeStruct((_BATCH, _KNN), jnp.int32),
    ],
    mesh=plsc.VectorSubcoreMesh(core_axis_name="c", subcore_axis_name="s"),
    compiler_params=pltpu.CompilerParams(needs_layout_passes=False),
    scratch_types=[
        pltpu.VMEM((_RPW, 1), jnp.int32),
        pltpu.VMEM((2, 1, _N_ITEMS), jnp.float32),
        pltpu.VMEM((_CAP,), jnp.float32),
        pltpu.VMEM((_CAP,), jnp.int32),
        pltpu.VMEM((_TOPB,), jnp.float32),
        pltpu.VMEM((_TOPB,), jnp.int32),
        pltpu.VMEM((_RPW, _KNN), jnp.float32),
        pltpu.VMEM((_RPW, _KNN), jnp.int32),
        pltpu.SemaphoreType.DMA,
        pltpu.SemaphoreType.DMA,
    ],
)(_topk_body)


def kernel(item_id, sims):
    iid = item_id.astype(jnp.int32).reshape(_BATCH, 1)
    vals, idxs = _sc_topk(iid, sims)
    return vals, idxs


# speedup vs baseline: 187.0689x; 1.5357x over previous
import functools

import jax
import jax.numpy as jnp
from jax import lax
from jax.experimental import pallas as pl
from jax.experimental.pallas import tpu as pltpu
from jax.experimental.pallas import tpu_sc as plsc

_N_ITEMS = 16384
_KNN = 100
_BATCH = 4096
_L = 16
_NC = 2
_NS = 16
_NW = _NC * _NS
_RPW = _BATCH // _NW
_NV = _N_ITEMS // _L
_GRP = 8
_NG = _NV // _GRP
_TOPB = 112
_NSLOT = _TOPB // _L
_CAP = _N_ITEMS + _L
_NEG = -3.4028234663852886e38
_THRESH = 2.21


def _topk_body(iid_hbm, sims_hbm, ov_hbm, oi_hbm,
               ids_v, row_v, ci_v, tv_v, ti_v, sv_v, si_v,
               sem0, sem1):
    wid = lax.axis_index("s") * _NC + lax.axis_index("c")
    base = wid * _RPW
    pltpu.sync_copy(iid_hbm.at[pl.ds(base, _RPW)], ids_v)
    iota = lax.iota(jnp.int32, _L)
    even = (iota & 1) == 0
    perm_a = iota ^ 1
    perm_b = jnp.clip(jnp.where(even, iota - 1, iota + 1), 0, _L - 1)

    def _bcast(x, lane):
        return x.at[jnp.full((_L,), lane, jnp.int32)].get(
            mode="promise_in_bounds")

    def process_row(buf, rr):
        for j in range(_NSLOT):
            tv_v[pl.ds(_L * j, _L)] = jnp.full((_L,), _NEG, jnp.float32)
            ti_v[pl.ds(_L * j, _L)] = jnp.zeros((_L,), jnp.int32)

        def pa_body(i, pos):
            b0 = i * (_GRP * _L)
            vs = [row_v[buf, 0, pl.ds(b0 + _L * q, _L)] for q in range(_GRP)]
            ms = [v >= _THRESH for v in vs]
            cs = [plsc.all_reduce_population_count(m)[0] for m in ms]
            p = pos
            for q in range(_GRP):
                plsc.store_compressed(ci_v.at[pl.ds(p, _L)],
                                      iota + (b0 + _L * q), mask=ms[q])
                p = p + cs[q]
            return p

        n = lax.fori_loop(0, _NG, pa_body, jnp.int32(0))

        def fallback():
            def cp(i, _):
                ci_v[pl.ds(i * _L, _L)] = iota + i * _L
                return 0

            lax.fori_loop(0, _NV, cp, 0)
            return jnp.int32(_N_ITEMS)

        n2 = lax.cond(n < _TOPB, fallback, lambda: n)
        nvec = (n2 + _L - 1) // _L
        bufv = jnp.full((_L,), buf, jnp.int32)
        zerov = jnp.zeros((_L,), jnp.int32)

        def pb_body(i, thrv):
            vi = ci_v[pl.ds(i * _L, _L)]
            valid = (iota + i * _L) < n2
            vi = jnp.where(valid, vi, zerov)
            v = plsc.load_gather(row_v, [bufv, zerov, vi])
            v = jnp.where(valid, v, _NEG)
            m = v > thrv

            def do_insert():
                ck, cvp = plsc.sort_key_val(v, vi, descending=True)
                cur_k, cur_v = ck, cvp
                last_hk = None
                for j in range(_NSLOT):
                    ak = tv_v[pl.ds(_L * j, _L)]
                    ai = ti_v[pl.ds(_L * j, _L)]
                    rk = jnp.flip(cur_k, 0)
                    rv = jnp.flip(cur_v, 0)
                    c = ak >= rk
                    hk = jnp.where(c, ak, rk)
                    hv = jnp.where(c, ai, rv)
                    lk = jnp.where(c, rk, ak)
                    lv = jnp.where(c, rv, ai)
                    hk, hv = plsc.sort_key_val(hk, hv, descending=True)
                    tv_v[pl.ds(_L * j, _L)] = hk
                    ti_v[pl.ds(_L * j, _L)] = hv
                    if j < _NSLOT - 1:
                        cur_k, cur_v = plsc.sort_key_val(lk, lv,
                                                         descending=True)
                    last_hk = hk
                return _bcast(last_hk, _L - 1)

            return lax.cond(jnp.any(m), do_insert, lambda: thrv)

        thr0 = jnp.full((_L,), _NEG, jnp.float32)
        lax.fori_loop(0, nvec, pb_body, thr0)

        vs = [tv_v[pl.ds(_L * j, _L)] for j in range(_NSLOT)]
        ix = [ti_v[pl.ds(_L * j, _L)] for j in range(_NSLOT)]
        for phase in range(4):
            if phase % 2 == 0:
                for j in range(_NSLOT):
                    pv = vs[j].at[perm_a].get(mode="promise_in_bounds")
                    pi = ix[j].at[perm_a].get(mode="promise_in_bounds")
                    veq = vs[j] == pv
                    upd = jnp.where(even, jnp.minimum(ix[j], pi),
                                    jnp.maximum(ix[j], pi))
                    ix[j] = jnp.where(veq, upd, ix[j])
            else:
                ix0 = list(ix)
                for j in range(_NSLOT):
                    pv = vs[j].at[perm_b].get(mode="promise_in_bounds")
                    pi = ix0[j].at[perm_b].get(mode="promise_in_bounds")
                    elig = jnp.ones((_L,), jnp.bool_)
                    if j > 0:
                        at0 = iota == 0
                        pv = jnp.where(at0, _bcast(vs[j - 1], _L - 1), pv)
                        pi = jnp.where(at0, _bcast(ix0[j - 1], _L - 1), pi)
                    else:
                        elig = elig & (iota != 0)
                    if j < _NSLOT - 1:
                        at15 = iota == _L - 1
                        pv = jnp.where(at15, _bcast(vs[j + 1], 0), pv)
                        pi = jnp.where(at15, _bcast(ix0[j + 1], 0), pi)
                    else:
                        elig = elig & (iota != _L - 1)
                    veq = (vs[j] == pv) & elig
                    upd = jnp.where(~even, jnp.minimum(ix0[j], pi),
                                    jnp.maximum(ix0[j], pi))
                    ix[j] = jnp.where(veq, upd, ix0[j])

        ti_v[pl.ds(_L * (_NSLOT - 2), _L)] = ix[_NSLOT - 2]
        ti_v[pl.ds(_L * (_NSLOT - 1), _L)] = ix[_NSLOT - 1]
        for j in range(_KNN // _L):
            sv_v[rr, pl.ds(_L * j, _L)] = vs[j]
            si_v[rr, pl.ds(_L * j, _L)] = ix[j]
        sv_v[rr, pl.ds(_KNN - _L, _L)] = tv_v[pl.ds(_KNN - _L, _L)]
        si_v[rr, pl.ds(_KNN - _L, _L)] = ti_v[pl.ds(_KNN - _L, _L)]

    def issue(rr, buf, sem):
        pltpu.async_copy(sims_hbm.at[ids_v.at[rr]], row_v.at[buf], sem)

    def wait(rr, buf, sem):
        pltpu.make_async_copy(sims_hbm.at[ids_v.at[rr]],
                              row_v.at[buf], sem).wait()

    issue(0, 0, sem0)

    def pair_body(k, carry):
        r0 = 2 * k
        issue(r0 + 1, 1, sem1)
        wait(r0, 0, sem0)
        process_row(0, r0)

        @pl.when(k < _RPW // 2 - 1)
        def _():
            issue(r0 + 2, 0, sem0)

        wait(r0 + 1, 1, sem1)
        process_row(1, r0 + 1)
        return carry

    lax.fori_loop(0, _RPW // 2, pair_body, 0)

    pltpu.sync_copy(sv_v, ov_hbm.at[pl.ds(base, _RPW)])
    pltpu.sync_copy(si_v, oi_hbm.at[pl.ds(base, _RPW)])


_sc_topk = functools.partial(
    pl.kernel,
    out_type=[
        jax.ShapeDtypeStruct((_BATCH, _KNN), jnp.float32),
        jax.ShapeDtypeStruct((_BATCH, _KNN), jnp.int32),
    ],
    mesh=plsc.VectorSubcoreMesh(core_axis_name="c", subcore_axis_name="s"),
    compiler_params=pltpu.CompilerParams(needs_layout_passes=False),
    scratch_types=[
        pltpu.VMEM((_RPW, 1), jnp.int32),
        pltpu.VMEM((2, 1, _N_ITEMS), jnp.float32),
        pltpu.VMEM((_CAP,), jnp.int32),
        pltpu.VMEM((_TOPB,), jnp.float32),
        pltpu.VMEM((_TOPB,), jnp.int32),
        pltpu.VMEM((_RPW, _KNN), jnp.float32),
        pltpu.VMEM((_RPW, _KNN), jnp.int32),
        pltpu.SemaphoreType.DMA,
        pltpu.SemaphoreType.DMA,
    ],
)(_topk_body)


def kernel(item_id, sims):
    iid = item_id.astype(jnp.int32).reshape(_BATCH, 1)
    vals, idxs = _sc_topk(iid, sims)
    return vals, idxs
